# baseline (speedup 1.0000x reference)
import functools

import jax
import jax.numpy as jnp
from jax import lax
from jax.experimental import pallas as pl
from jax.experimental.pallas import tpu as pltpu
from jax.experimental.pallas import tpu_sc as plsc

_B, _S, _D = 4, 4096, 1024
_NC, _NS = 2, 16
_NW = _NC * _NS
_ROWS_PW = (_B * _S) // _NW
_C = 16
_NCH = _ROWS_PW // _C
_CE = _C * _D
_LANES = 16


def _sc_body(in_hbm, pos_hbm, out_hbm, x_v, p_v, sem_x, sem_p):
    wid = lax.axis_index("s") * _NC + lax.axis_index("c")
    base = wid * _ROWS_PW * _D
    pbase = (wid % (_NW // _B)) * _ROWS_PW * _D

    def chunk(g, carry):
        off = base + g * _CE
        poff = pbase + g * _CE
        cp_x = pltpu.make_async_copy(in_hbm.at[pl.ds(off, _CE)], x_v, sem_x)
        cp_p = pltpu.make_async_copy(pos_hbm.at[pl.ds(poff, _CE)], p_v, sem_p)
        cp_x.start()
        cp_p.start()
        cp_x.wait()
        cp_p.wait()

        def vec(k, c):
            sl = pl.ds(k * _LANES, _LANES)
            x_v[sl] = x_v[sl] + p_v[sl]
            return c

        lax.fori_loop(0, _CE // _LANES, vec, 0, unroll=8)
        pltpu.sync_copy(x_v, out_hbm.at[pl.ds(off, _CE)])
        return carry

    lax.fori_loop(0, _NCH, chunk, 0)


_sc_add = functools.partial(
    pl.kernel,
    out_type=jax.ShapeDtypeStruct((_B * _S * _D,), jnp.float32),
    mesh=plsc.VectorSubcoreMesh(
        core_axis_name="c", subcore_axis_name="s",
        num_cores=_NC, num_subcores=_NS),
    scratch_types=[
        pltpu.VMEM((_CE,), jnp.float32),
        pltpu.VMEM((_CE,), jnp.float32),
        pltpu.SemaphoreType.DMA,
        pltpu.SemaphoreType.DMA,
    ],
)(_sc_body)


def kernel(inputs, pos_table):
    b, s, d = inputs.shape
    out = _sc_add(inputs.reshape(-1), pos_table.reshape(-1))
    return out.reshape(b, s, d)


# speedup vs baseline: 1.7280x; 1.7280x over previous
import functools

import jax
import jax.numpy as jnp
from jax import lax
from jax.experimental import pallas as pl
from jax.experimental.pallas import tpu as pltpu
from jax.experimental.pallas import tpu_sc as plsc

_B, _S, _D = 4, 4096, 1024
_NC, _NS = 2, 16
_NW = _NC * _NS
_ROWS_PW = (_B * _S) // _NW
_C = 8
_NCH = _ROWS_PW // _C
_CE = _C * _D
_LANES = 16
_NBUF = 4


def _sc_body(in_hbm, pos_hbm, out_hbm,
             x0, x1, x2, x3, p0, p1, p2, p3,
             sx0, sx1, sx2, sx3, sp0, sp1, sp2, sp3, so0, so1, so2, so3):
    xs = (x0, x1, x2, x3)
    ps = (p0, p1, p2, p3)
    sxs = (sx0, sx1, sx2, sx3)
    sps = (sp0, sp1, sp2, sp3)
    sos = (so0, so1, so2, so3)

    wid = lax.axis_index("s") * _NC + lax.axis_index("c")
    base = wid * _ROWS_PW * _D
    pbase = (wid % (_NW // _B)) * _ROWS_PW * _D

    def start_loads(g, j):
        pltpu.make_async_copy(
            in_hbm.at[pl.ds(base + g * _CE, _CE)], xs[j], sxs[j]).start()
        pltpu.make_async_copy(
            pos_hbm.at[pl.ds(pbase + g * _CE, _CE)], ps[j], sps[j]).start()

    def wait_loads(g, j):
        pltpu.make_async_copy(
            in_hbm.at[pl.ds(base + g * _CE, _CE)], xs[j], sxs[j]).wait()
        pltpu.make_async_copy(
            pos_hbm.at[pl.ds(pbase + g * _CE, _CE)], ps[j], sps[j]).wait()

    for j in range(_NBUF - 1):
        start_loads(j, j)

    def outer(i, carry):
        for j in range(_NBUF):
            g = i * _NBUF + j
            wait_loads(g, j)

            def vec(k, c):
                sl = pl.ds(k * _LANES, _LANES)
                plsc.addupdate(xs[j].at[sl], ps[j][sl])
                return c

            lax.fori_loop(0, _CE // _LANES, vec, 0, unroll=16)

            out_cp = pltpu.make_async_copy(
                xs[j], out_hbm.at[pl.ds(base + g * _CE, _CE)], sos[j])
            out_cp.start()

            jn = (j + _NBUF - 1) % _NBUF

            @pl.when(jnp.logical_and(g >= 1, g + _NBUF - 1 < _NCH))
            def _():
                pltpu.make_async_copy(
                    xs[jn],
                    out_hbm.at[pl.ds(base + (g - 1) * _CE, _CE)],
                    sos[jn]).wait()

            @pl.when(g + _NBUF - 1 < _NCH)
            def _():
                start_loads(g + _NBUF - 1, jn)
        return carry

    lax.fori_loop(0, _NCH // _NBUF, outer, 0)

    for j in range(_NBUF):
        g = _NCH - _NBUF + j
        pltpu.make_async_copy(
            xs[j], out_hbm.at[pl.ds(base + g * _CE, _CE)], sos[j]).wait()


_sc_add = functools.partial(
    pl.kernel,
    out_type=jax.ShapeDtypeStruct((_B * _S * _D,), jnp.float32),
    mesh=plsc.VectorSubcoreMesh(
        core_axis_name="c", subcore_axis_name="s",
        num_cores=_NC, num_subcores=_NS),
    scratch_types=(
        [pltpu.VMEM((_CE,), jnp.float32)] * (2 * _NBUF)
        + [pltpu.SemaphoreType.DMA] * (3 * _NBUF)
    ),
)(_sc_body)


def kernel(inputs, pos_table):
    b, s, d = inputs.shape
    out = _sc_add(inputs.reshape(-1), pos_table.reshape(-1))
    return out.reshape(b, s, d)


# speedup vs baseline: 2.5903x; 1.4990x over previous
import functools

import jax
import jax.numpy as jnp
from jax import lax
from jax.experimental import pallas as pl
from jax.experimental.pallas import tpu as pltpu
from jax.experimental.pallas import tpu_sc as plsc

_B, _S, _D = 4, 4096, 1024
_BS = 2048
_SC_ROWS = 2048
_SC_IN_OFF = (3 * _S + (_S - _SC_ROWS)) * _D
_SC_POS_OFF = (_S - _SC_ROWS) * _D

_NC, _NS = 2, 16
_NW = _NC * _NS
_ROWS_PW = _SC_ROWS // _NW
_C = 8
_NCH = _ROWS_PW // _C
_CE = _C * _D
_LANES = 16
_NBUF = 4


def _tc_add_body(x_ref, p_ref, o_ref):
    o_ref[...] = x_ref[...] + p_ref[...]


def _tc_add(inputs, pos_table):
    return pl.pallas_call(
        _tc_add_body,
        grid=(7,),
        in_specs=[
            pl.BlockSpec((1, _BS, _D), lambda k: (k % 4, k // 4, 0)),
            pl.BlockSpec((_BS, _D), lambda k: (k // 4, 0)),
        ],
        out_specs=pl.BlockSpec((1, _BS, _D), lambda k: (k % 4, k // 4, 0)),
        out_shape=jax.ShapeDtypeStruct((_B, _S, _D), inputs.dtype),
    )(inputs, pos_table)


def _sc_body(in_hbm, pos_hbm, out_hbm,
             x0, x1, x2, x3, p0, p1, p2, p3,
             sx0, sx1, sx2, sx3, sp0, sp1, sp2, sp3, so0, so1, so2, so3):
    xs = (x0, x1, x2, x3)
    ps = (p0, p1, p2, p3)
    sxs = (sx0, sx1, sx2, sx3)
    sps = (sp0, sp1, sp2, sp3)
    sos = (so0, so1, so2, so3)

    wid = lax.axis_index("s") * _NC + lax.axis_index("c")
    base = _SC_IN_OFF + wid * _ROWS_PW * _D
    obase = wid * _ROWS_PW * _D
    pbase = _SC_POS_OFF + wid * _ROWS_PW * _D

    def start_loads(g, j):
        pltpu.make_async_copy(
            in_hbm.at[pl.ds(base + g * _CE, _CE)], xs[j], sxs[j]).start()
        pltpu.make_async_copy(
            pos_hbm.at[pl.ds(pbase + g * _CE, _CE)], ps[j], sps[j]).start()

    def wait_loads(g, j):
        pltpu.make_async_copy(
            in_hbm.at[pl.ds(base + g * _CE, _CE)], xs[j], sxs[j]).wait()
        pltpu.make_async_copy(
            pos_hbm.at[pl.ds(pbase + g * _CE, _CE)], ps[j], sps[j]).wait()

    for j in range(_NBUF - 1):
        start_loads(j, j)

    def outer(i, carry):
        for j in range(_NBUF):
            g = i * _NBUF + j
            wait_loads(g, j)

            def vec(k, c):
                sl = pl.ds(k * _LANES, _LANES)
                plsc.addupdate(xs[j].at[sl], ps[j][sl])
                return c

            lax.fori_loop(0, _CE // _LANES, vec, 0, unroll=16)

            pltpu.make_async_copy(
                xs[j], out_hbm.at[pl.ds(obase + g * _CE, _CE)], sos[j]).start()

            jn = (j + _NBUF - 1) % _NBUF

            @pl.when(jnp.logical_and(g >= 1, g + _NBUF - 1 < _NCH))
            def _():
                pltpu.make_async_copy(
                    xs[jn],
                    out_hbm.at[pl.ds(obase + (g - 1) * _CE, _CE)],
                    sos[jn]).wait()

            @pl.when(g + _NBUF - 1 < _NCH)
            def _():
                start_loads(g + _NBUF - 1, jn)
        return carry

    lax.fori_loop(0, _NCH // _NBUF, outer, 0)

    for j in range(_NBUF):
        g = _NCH - _NBUF + j
        pltpu.make_async_copy(
            xs[j], out_hbm.at[pl.ds(obase + g * _CE, _CE)], sos[j]).wait()


_sc_add = functools.partial(
    pl.kernel,
    out_type=jax.ShapeDtypeStruct((_SC_ROWS * _D,), jnp.float32),
    mesh=plsc.VectorSubcoreMesh(
        core_axis_name="c", subcore_axis_name="s",
        num_cores=_NC, num_subcores=_NS),
    scratch_types=(
        [pltpu.VMEM((_CE,), jnp.float32)] * (2 * _NBUF)
        + [pltpu.SemaphoreType.DMA] * (3 * _NBUF)
    ),
)(_sc_body)


def kernel(inputs, pos_table):
    sc_part = _sc_add(inputs.reshape(-1), pos_table.reshape(-1))
    tc_out = _tc_add(inputs, pos_table)
    return lax.dynamic_update_slice(
        tc_out, sc_part.reshape(1, _SC_ROWS, _D), (3, _S - _SC_ROWS, 0))


# speedup vs baseline: 5.2613x; 2.0312x over previous
import functools

import jax
import jax.numpy as jnp
from jax import lax
from jax.experimental import pallas as pl
from jax.experimental.pallas import tpu as pltpu
from jax.experimental.pallas import tpu_sc as plsc

_B, _S, _D = 4, 4096, 1024
_BS = 2048
_SC_ROWS = 2048
_SC_IN_OFF = (3 * _S + (_S - _SC_ROWS)) * _D
_SC_POS_OFF = (_S - _SC_ROWS) * _D

_NC, _NS = 2, 16
_NW = _NC * _NS
_ROWS_PW = _SC_ROWS // _NW
_C = 8
_NCH = _ROWS_PW // _C
_CE = _C * _D
_LANES = 16
_NBUF = 4


def _tc_add_body(x_ref, p_ref, o_ref):
    o_ref[...] = x_ref[...] + p_ref[...]


def _tc_add(inputs, pos_table):
    return pl.pallas_call(
        _tc_add_body,
        grid=(7,),
        in_specs=[
            pl.BlockSpec((1, _BS, _D), lambda k: (k % 4, k // 4, 0)),
            pl.BlockSpec((_BS, _D), lambda k: (k // 4, 0)),
        ],
        out_specs=pl.BlockSpec((1, _BS, _D), lambda k: (k % 4, k // 4, 0)),
        out_shape=jax.ShapeDtypeStruct((_B, _S, _D), inputs.dtype),
    )(inputs, pos_table)


def _sc_body(in_hbm, pos_hbm, out_hbm,
             x0, x1, x2, x3, p0, p1, p2, p3,
             sx0, sx1, sx2, sx3, sp0, sp1, sp2, sp3, so0, so1, so2, so3):
    xs = (x0, x1, x2, x3)
    ps = (p0, p1, p2, p3)
    sxs = (sx0, sx1, sx2, sx3)
    sps = (sp0, sp1, sp2, sp3)
    sos = (so0, so1, so2, so3)

    wid = lax.axis_index("s") * _NC + lax.axis_index("c")
    row0 = (_S - _SC_ROWS) + wid * _ROWS_PW
    orow0 = wid * _ROWS_PW

    def start_loads(g, j):
        pltpu.make_async_copy(
            in_hbm.at[3, pl.ds(row0 + g * _C, _C), :], xs[j], sxs[j]).start()
        pltpu.make_async_copy(
            pos_hbm.at[pl.ds(row0 + g * _C, _C), :], ps[j], sps[j]).start()

    def wait_loads(g, j):
        pltpu.make_async_copy(
            in_hbm.at[3, pl.ds(row0 + g * _C, _C), :], xs[j], sxs[j]).wait()
        pltpu.make_async_copy(
            pos_hbm.at[pl.ds(row0 + g * _C, _C), :], ps[j], sps[j]).wait()

    for j in range(_NBUF - 1):
        start_loads(j, j)

    def outer(i, carry):
        for j in range(_NBUF):
            g = i * _NBUF + j
            wait_loads(g, j)

            def vec(k, c):
                r = k // (_D // _LANES)
                sl = pl.ds((k % (_D // _LANES)) * _LANES, _LANES)
                plsc.addupdate(xs[j].at[r, sl], ps[j][r, sl])
                return c

            lax.fori_loop(0, _CE // _LANES, vec, 0, unroll=16)

            pltpu.make_async_copy(
                xs[j], out_hbm.at[pl.ds(orow0 + g * _C, _C), :],
                sos[j]).start()

            jn = (j + _NBUF - 1) % _NBUF

            @pl.when(jnp.logical_and(g >= 1, g + _NBUF - 1 < _NCH))
            def _():
                pltpu.make_async_copy(
                    xs[jn],
                    out_hbm.at[pl.ds(orow0 + (g - 1) * _C, _C), :],
                    sos[jn]).wait()

            @pl.when(g + _NBUF - 1 < _NCH)
            def _():
                start_loads(g + _NBUF - 1, jn)
        return carry

    lax.fori_loop(0, _NCH // _NBUF, outer, 0)

    for j in range(_NBUF):
        g = _NCH - _NBUF + j
        pltpu.make_async_copy(
            xs[j], out_hbm.at[pl.ds(orow0 + g * _C, _C), :], sos[j]).wait()


_sc_add = functools.partial(
    pl.kernel,
    out_type=jax.ShapeDtypeStruct((_SC_ROWS, _D), jnp.float32),
    mesh=plsc.VectorSubcoreMesh(
        core_axis_name="c", subcore_axis_name="s",
        num_cores=_NC, num_subcores=_NS),
    scratch_types=(
        [pltpu.VMEM((_C, _D), jnp.float32)] * (2 * _NBUF)
        + [pltpu.SemaphoreType.DMA] * (3 * _NBUF)
    ),
)(_sc_body)


def kernel(inputs, pos_table):
    sc_part = _sc_add(inputs, pos_table)
    tc_out = _tc_add(inputs, pos_table)
    return lax.dynamic_update_slice(
        tc_out, sc_part[None], (3, _S - _SC_ROWS, 0))


# speedup vs baseline: 8.2586x; 1.5697x over previous
import jax
import jax.numpy as jnp
from jax.experimental import pallas as pl


_BS = 2048


def _add_kernel(x_ref, p_ref, o_ref):
    o_ref[...] = x_ref[...] + p_ref[...]


def kernel(inputs, pos_table):
    b, s, d = inputs.shape
    grid = (s // _BS, b)
    return pl.pallas_call(
        _add_kernel,
        grid=grid,
        in_specs=[
            pl.BlockSpec((1, _BS, d), lambda j, i: (i, j, 0)),
            pl.BlockSpec((_BS, d), lambda j, i: (j, 0)),
        ],
        out_specs=pl.BlockSpec((1, _BS, d), lambda j, i: (i, j, 0)),
        out_shape=jax.ShapeDtypeStruct((b, s, d), inputs.dtype),
    )(inputs, pos_table)


# speedup vs baseline: 8.4342x; 1.0213x over previous
import jax
import jax.numpy as jnp
from jax.experimental import pallas as pl
from jax.experimental.pallas import tpu as pltpu

_B, _S, _D = 4, 4096, 1024
_R = 2048
_NCH = (_B * _S) // _R
_NBUF = 4


def _body(in_hbm, pos_hbm, out_hbm, b0, b1, b2, b3, pos_v,
          sp, si0, si1, si2, si3, so0, so1, so2, so3):
    bufs = (b0, b1, b2, b3)
    sis = (si0, si1, si2, si3)
    sos = (so0, so1, so2, so3)

    ins = [pltpu.make_async_copy(
        in_hbm.at[pl.ds(c * _R, _R), :], bufs[c % _NBUF], sis[c % _NBUF])
        for c in range(_NCH)]
    outs = [pltpu.make_async_copy(
        bufs[c % _NBUF], out_hbm.at[pl.ds(c * _R, _R), :], sos[c % _NBUF])
        for c in range(_NCH)]

    pos_cp = pltpu.make_async_copy(pos_hbm, pos_v, sp)
    pos_cp.start()
    for c in range(_NBUF - 1):
        ins[c].start()
    pos_cp.wait()

    for c in range(_NCH):
        j = c % _NBUF
        ins[c].wait()
        p0 = (c % (_S // _R)) * _R
        bufs[j][...] = bufs[j][...] + pos_v[pl.ds(p0, _R), :]
        outs[c].start()
        if c >= 1 and c + _NBUF - 1 < _NCH:
            outs[c - 1].wait()
        if c + _NBUF - 1 < _NCH:
            ins[c + _NBUF - 1].start()

    for c in range(_NCH - _NBUF, _NCH):
        outs[c].wait()


def kernel(inputs, pos_table):
    b, s, d = inputs.shape
    out = pl.pallas_call(
        _body,
        in_specs=[
            pl.BlockSpec(memory_space=pl.ANY),
            pl.BlockSpec(memory_space=pl.ANY),
        ],
        out_specs=pl.BlockSpec(memory_space=pl.ANY),
        out_shape=jax.ShapeDtypeStruct((b * s, d), inputs.dtype),
        scratch_shapes=(
            [pltpu.VMEM((_R, _D), jnp.float32)] * _NBUF
            + [pltpu.VMEM((_S, _D), jnp.float32)]
            + [pltpu.SemaphoreType.DMA] * (2 * _NBUF + 1)
        ),
    )(inputs.reshape(b * s, d), pos_table)
    return out.reshape(b, s, d)


# speedup vs baseline: 8.4522x; 1.0021x over previous
import jax
import jax.numpy as jnp
from jax.experimental import pallas as pl
from jax.experimental.pallas import tpu as pltpu

_B, _S, _D = 4, 4096, 1024
_R = 2048
_NCH = (_B * _S) // _R
_NBUF = 4


def _body(in_hbm, pos_hbm, out_hbm, b0, b1, b2, b3, pos_v,
          sp0, sp1, si0, si1, si2, si3, so0, so1, so2, so3):
    bufs = (b0, b1, b2, b3)
    sis = (si0, si1, si2, si3)
    sos = (so0, so1, so2, so3)

    ins = [pltpu.make_async_copy(
        in_hbm.at[pl.ds(c * _R, _R), :], bufs[c % _NBUF], sis[c % _NBUF])
        for c in range(_NCH)]
    outs = [pltpu.make_async_copy(
        bufs[c % _NBUF], out_hbm.at[pl.ds(c * _R, _R), :], sos[c % _NBUF])
        for c in range(_NCH)]
    pos_cps = [pltpu.make_async_copy(
        pos_hbm.at[pl.ds(h * _R, _R), :], pos_v.at[pl.ds(h * _R, _R), :], s)
        for h, s in ((0, sp0), (1, sp1))]

    pos_cps[0].start()
    ins[0].start()
    pos_cps[1].start()
    for c in range(1, _NBUF - 1):
        ins[c].start()

    for c in range(_NCH):
        j = c % _NBUF
        if c < len(pos_cps):
            pos_cps[c].wait()
        ins[c].wait()
        p0 = (c % (_S // _R)) * _R
        bufs[j][...] = bufs[j][...] + pos_v[pl.ds(p0, _R), :]
        outs[c].start()
        if c >= 1 and c + _NBUF - 1 < _NCH:
            outs[c - 1].wait()
        if c + _NBUF - 1 < _NCH:
            ins[c + _NBUF - 1].start()

    for c in range(_NCH - _NBUF, _NCH):
        outs[c].wait()


def kernel(inputs, pos_table):
    b, s, d = inputs.shape
    out = pl.pallas_call(
        _body,
        in_specs=[
            pl.BlockSpec(memory_space=pl.ANY),
            pl.BlockSpec(memory_space=pl.ANY),
        ],
        out_specs=pl.BlockSpec(memory_space=pl.ANY),
        out_shape=jax.ShapeDtypeStruct((b * s, d), inputs.dtype),
        scratch_shapes=(
            [pltpu.VMEM((_R, _D), jnp.float32)] * _NBUF
            + [pltpu.VMEM((_S, _D), jnp.float32)]
            + [pltpu.SemaphoreType.DMA] * (2 * _NBUF + 2)
        ),
    )(inputs.reshape(b * s, d), pos_table)
    return out.reshape(b, s, d)
